# baseline (speedup 1.0000x reference)
import functools

import jax
import jax.numpy as jnp
from jax.experimental import pallas as pl
from jax.experimental.pallas import tpu as pltpu

N_NODES = 10000
N_GRAPHS = 64
D = 128
CHUNK = 1000
NUM_CHUNKS = N_NODES // CHUNK


def _copy(x_hbm, xbuf, sem, k):
    return pltpu.make_async_copy(
        x_hbm.at[pl.ds(k * CHUNK, CHUNK), :],
        xbuf.at[k % 2],
        sem.at[k % 2])


def _fused_body(x_hbm, batch_ref, w1_ref, w2_ref, wend_ref, out_ref,
                xbuf, sem):
    w1 = w1_ref[...]
    w2 = w2_ref[...]
    wend_row = wend_ref[...].reshape(1, D)
    b_all = batch_ref[...].reshape(1, N_NODES)
    seg = jax.lax.broadcasted_iota(jnp.int32, (N_GRAPHS, CHUNK), 0)

    ones_row = jnp.ones((1, CHUNK), jnp.float32)
    _copy(x_hbm, xbuf, sem, 0).start()
    acc = jnp.zeros((N_GRAPHS, D), jnp.float32)
    counts = jnp.zeros((1, N_GRAPHS), jnp.float32)
    for k in range(NUM_CHUNKS):
        if k + 1 < NUM_CHUNKS:
            _copy(x_hbm, xbuf, sem, k + 1).start()
        _copy(x_hbm, xbuf, sem, k).wait()
        x = xbuf[k % 2]
        h = jax.lax.dot(x, w1, preferred_element_type=jnp.float32)
        h = jnp.maximum(h, 0.0)
        h = jax.lax.dot(h, w2, preferred_element_type=jnp.float32)
        h = jnp.maximum(h, 0.0)
        b = jax.lax.slice(b_all, (0, k * CHUNK), (1, (k + 1) * CHUNK))
        maskf = (b == seg).astype(jnp.float32)
        acc = acc + jax.lax.dot(maskf, h, preferred_element_type=jnp.float32)
        counts = counts + jax.lax.dot_general(
            ones_row, maskf, (((1,), (1,)), ((), ())),
            preferred_element_type=jnp.float32)

    sums = jax.lax.dot_general(
        wend_row, acc, (((1,), (1,)), ((), ())),
        preferred_element_type=jnp.float32)
    pooled = sums / jnp.maximum(counts, 1.0)
    out_ref[...] = jax.nn.sigmoid(pooled).reshape(N_GRAPHS)


@functools.partial(jax.jit, static_argnames=())
def _fused_call(x, batch, W1, W2, wend_flat):
    return pl.pallas_call(
        _fused_body,
        in_specs=[
            pl.BlockSpec(memory_space=pl.ANY),
            pl.BlockSpec((N_NODES,), lambda: (0,)),
            pl.BlockSpec((D, D), lambda: (0, 0)),
            pl.BlockSpec((D, D), lambda: (0, 0)),
            pl.BlockSpec((D,), lambda: (0,)),
        ],
        out_specs=pl.BlockSpec((N_GRAPHS,), lambda: (0,)),
        out_shape=jax.ShapeDtypeStruct((N_GRAPHS,), jnp.float32),
        scratch_shapes=[
            pltpu.VMEM((2, CHUNK, D), jnp.float32),
            pltpu.SemaphoreType.DMA((2,)),
        ],
    )(x, batch, W1, W2, wend_flat)


def kernel(x, edge_index, edge_attr, batch, W1, W2, Wend):
    del edge_index, edge_attr
    return _fused_call(x, batch, W1, W2, Wend.reshape(D)).reshape(N_GRAPHS, 1)


# speedup vs baseline: 1.4110x; 1.4110x over previous
import functools

import jax
import jax.numpy as jnp
from jax.experimental import pallas as pl
from jax.experimental.pallas import tpu as pltpu

N_NODES = 10000
N_GRAPHS = 64
D = 128
CHUNK = 2000
NUM_CHUNKS = N_NODES // CHUNK


def _copy(x_hbm, xbuf, sem, k):
    return pltpu.make_async_copy(
        x_hbm.at[pl.ds(k * CHUNK, CHUNK), :],
        xbuf.at[k % 2],
        sem.at[k % 2])


def _fused_body(x_hbm, batch_ref, w1_ref, w2_ref, wend_ref, out_ref,
                xbuf, sem):
    w1 = w1_ref[...]
    w2 = w2_ref[...]
    wend_row = wend_ref[...].reshape(1, D)
    b_all = batch_ref[...].reshape(1, N_NODES)
    seg = jax.lax.broadcasted_iota(jnp.int32, (N_GRAPHS, CHUNK), 0)

    ones_row = jnp.ones((1, CHUNK), jnp.float32)
    _copy(x_hbm, xbuf, sem, 0).start()
    acc = jnp.zeros((N_GRAPHS, D), jnp.float32)
    counts = jnp.zeros((1, N_GRAPHS), jnp.float32)
    for k in range(NUM_CHUNKS):
        if k + 1 < NUM_CHUNKS:
            _copy(x_hbm, xbuf, sem, k + 1).start()
        _copy(x_hbm, xbuf, sem, k).wait()
        x = xbuf[k % 2]
        h = jax.lax.dot(x, w1, preferred_element_type=jnp.float32)
        h = jnp.maximum(h, 0.0)
        h = jax.lax.dot(h, w2, preferred_element_type=jnp.float32)
        h = jnp.maximum(h, 0.0)
        b = jax.lax.slice(b_all, (0, k * CHUNK), (1, (k + 1) * CHUNK))
        maskf = (b == seg).astype(jnp.float32)
        acc = acc + jax.lax.dot(maskf, h, preferred_element_type=jnp.float32)
        counts = counts + jax.lax.dot_general(
            ones_row, maskf, (((1,), (1,)), ((), ())),
            preferred_element_type=jnp.float32)

    sums = jax.lax.dot_general(
        wend_row, acc, (((1,), (1,)), ((), ())),
        preferred_element_type=jnp.float32)
    pooled = sums / jnp.maximum(counts, 1.0)
    out_ref[...] = jax.nn.sigmoid(pooled).reshape(N_GRAPHS)


@functools.partial(jax.jit, static_argnames=())
def _fused_call(x, batch, W1, W2, wend_flat):
    return pl.pallas_call(
        _fused_body,
        in_specs=[
            pl.BlockSpec(memory_space=pl.ANY),
            pl.BlockSpec((N_NODES,), lambda: (0,)),
            pl.BlockSpec((D, D), lambda: (0, 0)),
            pl.BlockSpec((D, D), lambda: (0, 0)),
            pl.BlockSpec((D,), lambda: (0,)),
        ],
        out_specs=pl.BlockSpec((N_GRAPHS,), lambda: (0,)),
        out_shape=jax.ShapeDtypeStruct((N_GRAPHS,), jnp.float32),
        scratch_shapes=[
            pltpu.VMEM((2, CHUNK, D), jnp.float32),
            pltpu.SemaphoreType.DMA((2,)),
        ],
    )(x, batch, W1, W2, wend_flat)


def kernel(x, edge_index, edge_attr, batch, W1, W2, Wend):
    del edge_index, edge_attr
    return _fused_call(x, batch, W1, W2, Wend.reshape(D)).reshape(N_GRAPHS, 1)
